# baseline (speedup 1.0000x reference)
import functools

import jax
import jax.numpy as jnp
from jax import lax
from jax.experimental import pallas as pl
from jax.experimental.pallas import tpu as pltpu
from jax.experimental.pallas import tpu_sc as plsc

N_DIS = 2000
N_GENE = 20000
E = 500000
D_IN = 128
HID = 64
LAT = 32
OUT_D = 64

NC = 2
NS = 16
NW = NC * NS
CH = 128
CPW = 128
E_PAD = NW * CPW * CH
ND_PAD = 2048
NG_PAD = 20096
GR_D = ND_PAD // NS
GR_G = NG_PAD // NS

_F32 = jnp.float32


def _zero_fill(buf, rows, width):
    @pl.loop(0, rows)
    def _zrow(i):
        @pl.loop(0, width // 16)
        def _zcol(j):
            buf[i, pl.ds(j * 16, 16)] = jnp.zeros((16,), _F32)


def _zero_stripe(zbuf, acc_sp, s, gr):
    full, rem = gr // CH, gr % CH
    @pl.loop(0, full)
    def _z(t):
        pltpu.sync_copy(zbuf, acc_sp.at[pl.ds(s * gr + t * CH, CH)])
    if rem:
        pltpu.sync_copy(zbuf.at[pl.ds(0, rem)],
                        acc_sp.at[pl.ds(s * gr + full * CH, rem)])


def _sc_agg_pass(src2d, dst2d, tbl_d, tbl_g, width, nbuf, ist, ch):
    mesh = plsc.VectorSubcoreMesh(core_axis_name="c", subcore_axis_name="s")
    ncw = (E_PAD // NW) // ch
    out_type = [
        jax.ShapeDtypeStruct((NC, NG_PAD, width), _F32),
        jax.ShapeDtypeStruct((NC, ND_PAD, width), _F32),
    ]
    scratch = (
        [pltpu.VMEM_SHARED((NG_PAD, width), _F32),
         pltpu.VMEM_SHARED((ND_PAD, width), _F32),
         pltpu.VMEM((ist, ch), jnp.int32),
         pltpu.VMEM((ist, ch), jnp.int32)]
        + [pltpu.VMEM((ch, width), _F32)] * (2 * nbuf)
        + [pltpu.SemaphoreType.DMA] * (2 * nbuf)
    )

    def body(src_hbm, dst_hbm, tbl_d_hbm, tbl_g_hbm, agg_g_hbm, agg_d_hbm,
             agg_g_sp, agg_d_sp, idx_s, idx_d, *rest):
        rows_d = rest[0:nbuf]
        rows_g = rest[nbuf:2 * nbuf]
        sem_d = rest[2 * nbuf:3 * nbuf]
        sem_g = rest[3 * nbuf:4 * nbuf]

        c = lax.axis_index("c")
        s = lax.axis_index("s")
        w = c * NS + s

        zwin = rows_d[0].at[pl.ds(0, CH)]
        _zero_fill(zwin, CH, width)
        _zero_stripe(zwin, agg_g_sp, s, GR_G)
        _zero_stripe(zwin, agg_d_sp, s, GR_D)
        plsc.subcore_barrier()

        for st in range(ncw // ist):
            base = w * ncw + st * ist
            pltpu.sync_copy(src_hbm.at[pl.ds(base, ist)], idx_s)
            pltpu.sync_copy(dst_hbm.at[pl.ds(base, ist)], idx_d)

            for b in range(nbuf):
                pltpu.async_copy(tbl_d_hbm.at[idx_s.at[b]],
                                 rows_d[b], sem_d[b])
                pltpu.async_copy(tbl_g_hbm.at[idx_d.at[b]],
                                 rows_g[b], sem_g[b])

            @pl.loop(0, ist, step=nbuf)
            def _chunk(j):
                for b in range(nbuf):
                    k = j + b
                    pltpu.make_async_copy(
                        tbl_d_hbm.at[idx_s.at[k]], rows_d[b], sem_d[b]).wait()
                    pltpu.sync_copy(rows_d[b], agg_g_sp.at[idx_d.at[k]],
                                    add=True)

                    def _refill_d(b=b, k=k):
                        pltpu.async_copy(tbl_d_hbm.at[idx_s.at[k + nbuf]],
                                         rows_d[b], sem_d[b])
                    pl.when(k + nbuf < ist)(_refill_d)

                    pltpu.make_async_copy(
                        tbl_g_hbm.at[idx_d.at[k]], rows_g[b], sem_g[b]).wait()
                    pltpu.sync_copy(rows_g[b], agg_d_sp.at[idx_s.at[k]],
                                    add=True)

                    def _refill_g(b=b, k=k):
                        pltpu.async_copy(tbl_g_hbm.at[idx_d.at[k + nbuf]],
                                         rows_g[b], sem_g[b])
                    pl.when(k + nbuf < ist)(_refill_g)

        plsc.subcore_barrier()

        pltpu.sync_copy(agg_g_sp.at[pl.ds(s * GR_G, GR_G)],
                        agg_g_hbm.at[c, pl.ds(s * GR_G, GR_G)])
        pltpu.sync_copy(agg_d_sp.at[pl.ds(s * GR_D, GR_D)],
                        agg_d_hbm.at[c, pl.ds(s * GR_D, GR_D)])

    fn = pl.kernel(
        body, out_type=out_type, mesh=mesh, scratch_types=scratch,
        compiler_params=pltpu.CompilerParams(use_tc_tiling_on_sc=False))
    return fn(src2d, dst2d, tbl_d, tbl_g)


def _sc_count_pass(src2d, dst2d, nbuf):
    mesh = plsc.VectorSubcoreMesh(core_axis_name="c", subcore_axis_name="s")
    out_type = [
        jax.ShapeDtypeStruct((NC, NG_PAD, 16), _F32),
        jax.ShapeDtypeStruct((NC, ND_PAD, 16), _F32),
    ]
    scratch = (
        [pltpu.VMEM_SHARED((NG_PAD, 16), _F32),
         pltpu.VMEM_SHARED((ND_PAD, 16), _F32),
         pltpu.VMEM((CPW, CH), jnp.int32),
         pltpu.VMEM((CPW, CH), jnp.int32),
         pltpu.VMEM((CH, 16), _F32),
         pltpu.VMEM((CH, 16), _F32)]
        + [pltpu.SemaphoreType.DMA] * (2 * nbuf)
    )

    def body(src_hbm, dst_hbm, cnt_g_hbm, cnt_d_hbm,
             cnt_g_sp, cnt_d_sp, idx_s, idx_d, ones_v, zer16_v, *sems):
        sem_g = sems[0:nbuf]
        sem_d = sems[nbuf:2 * nbuf]

        c = lax.axis_index("c")
        s = lax.axis_index("s")
        w = c * NS + s

        _zero_fill(zer16_v, CH, 16)
        @pl.loop(0, CH)
        def _orow(i):
            ones_v[i, pl.ds(0, 16)] = jnp.ones((16,), _F32)

        _zero_stripe(zer16_v, cnt_g_sp, s, GR_G)
        _zero_stripe(zer16_v, cnt_d_sp, s, GR_D)
        plsc.subcore_barrier()

        pltpu.sync_copy(src_hbm.at[pl.ds(w * CPW, CPW)], idx_s)
        pltpu.sync_copy(dst_hbm.at[pl.ds(w * CPW, CPW)], idx_d)

        @pl.loop(0, CPW, step=nbuf)
        def _chunk(j):
            for b in range(nbuf):
                k = j + b

                def _drain(b=b, k=k):
                    pltpu.make_async_copy(
                        ones_v, cnt_g_sp.at[idx_d.at[k - nbuf]],
                        sem_g[b]).wait()
                    pltpu.make_async_copy(
                        ones_v, cnt_d_sp.at[idx_s.at[k - nbuf]],
                        sem_d[b]).wait()
                pl.when(j > 0)(_drain)

                pltpu.async_copy(ones_v, cnt_g_sp.at[idx_d.at[k]],
                                 sem_g[b], add=True)
                pltpu.async_copy(ones_v, cnt_d_sp.at[idx_s.at[k]],
                                 sem_d[b], add=True)

        for b in range(nbuf):
            k = CPW - nbuf + b
            pltpu.make_async_copy(
                ones_v, cnt_g_sp.at[idx_d.at[k]], sem_g[b]).wait()
            pltpu.make_async_copy(
                ones_v, cnt_d_sp.at[idx_s.at[k]], sem_d[b]).wait()

        plsc.subcore_barrier()

        pltpu.sync_copy(cnt_g_sp.at[pl.ds(s * GR_G, GR_G)],
                        cnt_g_hbm.at[c, pl.ds(s * GR_G, GR_G)])
        pltpu.sync_copy(cnt_d_sp.at[pl.ds(s * GR_D, GR_D)],
                        cnt_d_hbm.at[c, pl.ds(s * GR_D, GR_D)])

    fn = pl.kernel(
        body, out_type=out_type, mesh=mesh, scratch_types=scratch,
        compiler_params=pltpu.CompilerParams(use_tc_tiling_on_sc=False))
    return fn(src2d, dst2d)


def _dot(a, b):
    return jnp.dot(a, b, preferred_element_type=_F32,
                   precision=lax.Precision.HIGHEST)


def _split_mm_body(x_ref, w_ref, o1_ref, o2_ref, *, half):
    acc = _dot(x_ref[...], w_ref[...])
    o1_ref[...] = acc[:, :half]
    o2_ref[...] = acc[:, half:]


def _proj_pair(x, w_cat, half, blk):
    n, d = x.shape
    return pl.pallas_call(
        functools.partial(_split_mm_body, half=half),
        grid=(pl.cdiv(n, blk),),
        in_specs=[pl.BlockSpec((blk, d), lambda i: (i, 0)),
                  pl.BlockSpec(w_cat.shape, lambda i: (0, 0))],
        out_specs=[pl.BlockSpec((blk, half), lambda i: (i, 0)),
                   pl.BlockSpec((blk, half), lambda i: (i, 0))],
        out_shape=[jax.ShapeDtypeStruct((n, half), _F32)] * 2,
    )(x, w_cat)


def _sage_epilogue_body(a_ref, c_ref, r_ref, b_ref, w_ref, o1_ref, o2_ref,
                        *, half):
    cnt = c_ref[0, :, 0:1] + c_ref[1, :, 0:1]
    agg = (a_ref[0, :, :] + a_ref[1, :, :]) / jnp.clip(cnt, 1.0, None)
    h = agg + b_ref[...] + r_ref[...]
    acc = _dot(h, w_ref[...])
    o1_ref[...] = acc[:, :half]
    o2_ref[...] = acc[:, half:]


def _sage_epilogue(agg_p, cnt_p, r, b, w_cat, half, blk):
    n, hid = r.shape
    return pl.pallas_call(
        functools.partial(_sage_epilogue_body, half=half),
        grid=(pl.cdiv(n, blk),),
        in_specs=[pl.BlockSpec((2, blk, hid), lambda i: (0, i, 0)),
                  pl.BlockSpec((2, blk, 16), lambda i: (0, i, 0)),
                  pl.BlockSpec((blk, hid), lambda i: (i, 0)),
                  pl.BlockSpec(b.shape, lambda i: (0, 0)),
                  pl.BlockSpec(w_cat.shape, lambda i: (0, 0))],
        out_specs=[pl.BlockSpec((blk, half), lambda i: (i, 0)),
                   pl.BlockSpec((blk, half), lambda i: (i, 0))],
        out_shape=[jax.ShapeDtypeStruct((n, half), _F32)] * 2,
    )(agg_p, cnt_p, r, b, w_cat)


def _latent_body(a_ref, c_ref, r_ref, b_ref, wmu_ref, bmu_ref, wlv_ref,
                 blv_ref, eps_ref, o_ref):
    cnt = c_ref[0, :, 0:1] + c_ref[1, :, 0:1]
    h = ((a_ref[0, :, :] + a_ref[1, :, :]) / jnp.clip(cnt, 1.0, None)
         + b_ref[...] + r_ref[...])
    mu = _dot(h, wmu_ref[...]) + bmu_ref[...]
    lv = _dot(h, wlv_ref[...]) + blv_ref[...]
    o_ref[...] = mu + eps_ref[...] * jnp.exp(lv)


def _latent_d(agg_p, cnt_p, r, b, wmu, bmu, wlv, blv, eps):
    return pl.pallas_call(
        _latent_body,
        grid=(1,),
        in_specs=[pl.BlockSpec((2, N_DIS, LAT), lambda i: (0, 0, 0)),
                  pl.BlockSpec((2, N_DIS, 16), lambda i: (0, 0, 0)),
                  pl.BlockSpec((N_DIS, LAT), lambda i: (0, 0)),
                  pl.BlockSpec(b.shape, lambda i: (0, 0)),
                  pl.BlockSpec(wmu.shape, lambda i: (0, 0)),
                  pl.BlockSpec(bmu.shape, lambda i: (0, 0)),
                  pl.BlockSpec(wlv.shape, lambda i: (0, 0)),
                  pl.BlockSpec(blv.shape, lambda i: (0, 0)),
                  pl.BlockSpec(eps.shape, lambda i: (0, 0))],
        out_specs=pl.BlockSpec((N_DIS, OUT_D), lambda i: (0, 0)),
        out_shape=jax.ShapeDtypeStruct((N_DIS, OUT_D), _F32),
    )(agg_p, cnt_p, r, b, wmu, bmu, wlv, blv, eps)


def _decode_body(zd_ref, a_ref, c_ref, r_ref, b_ref, wmu_ref, bmu_ref,
                 wlv_ref, blv_ref, eps_ref, o_ref):
    cnt = c_ref[0, :, 0:1] + c_ref[1, :, 0:1]
    h = ((a_ref[0, :, :] + a_ref[1, :, :]) / jnp.clip(cnt, 1.0, None)
         + b_ref[...] + r_ref[...])
    mu = _dot(h, wmu_ref[...]) + bmu_ref[...]
    lv = _dot(h, wlv_ref[...]) + blv_ref[...]
    zg = mu + eps_ref[...] * jnp.exp(lv)
    o_ref[...] = lax.dot_general(zd_ref[...], zg, (((1,), (1,)), ((), ())),
                                 preferred_element_type=_F32,
                                 precision=lax.Precision.HIGHEST)


def _decode(zd, agg_p, cnt_p, r, b, wmu, bmu, wlv, blv, eps, blk):
    return pl.pallas_call(
        _decode_body,
        grid=(pl.cdiv(NG_PAD, blk),),
        in_specs=[pl.BlockSpec((N_DIS, OUT_D), lambda i: (0, 0)),
                  pl.BlockSpec((2, blk, LAT), lambda i: (0, i, 0)),
                  pl.BlockSpec((2, blk, 16), lambda i: (0, i, 0)),
                  pl.BlockSpec((blk, LAT), lambda i: (i, 0)),
                  pl.BlockSpec(b.shape, lambda i: (0, 0)),
                  pl.BlockSpec(wmu.shape, lambda i: (0, 0)),
                  pl.BlockSpec(bmu.shape, lambda i: (0, 0)),
                  pl.BlockSpec(wlv.shape, lambda i: (0, 0)),
                  pl.BlockSpec(blv.shape, lambda i: (0, 0)),
                  pl.BlockSpec((blk, OUT_D), lambda i: (i, 0))],
        out_specs=pl.BlockSpec((N_DIS, blk), lambda i: (0, i)),
        out_shape=jax.ShapeDtypeStruct((N_DIS, N_GENE), _F32),
    )(zd, agg_p, cnt_p, r, b, wmu, bmu, wlv, blv, eps)


def kernel(x_disease, x_gene, edge_src, edge_dst, params):
    p = params

    src = edge_src.astype(jnp.int32)
    dst = edge_dst.astype(jnp.int32)
    pad_i = jnp.arange(E_PAD - E, dtype=jnp.int32)
    src2d = jnp.concatenate(
        [src, N_DIS + pad_i % (ND_PAD - N_DIS)]).reshape(NW * CPW, CH)
    dst2d = jnp.concatenate(
        [dst, N_GENE + pad_i % (NG_PAD - N_GENE)]).reshape(NW * CPW, CH)
    xd = jnp.concatenate(
        [x_disease, jnp.zeros((ND_PAD - N_DIS, D_IN), _F32)])
    xg = jnp.concatenate(
        [x_gene, jnp.zeros((NG_PAD - N_GENE, D_IN), _F32)])

    b1dg = p['b1dg'].reshape(1, HID)
    b1gd = p['b1gd'].reshape(1, HID)
    b2dg = p['b2dg'].reshape(1, LAT)
    b2gd = p['b2gd'].reshape(1, LAT)
    bmu_d = p['bmu_d'].reshape(1, OUT_D)
    blv_d = p['blv_d'].reshape(1, OUT_D)
    bmu_g = p['bmu_g'].reshape(1, OUT_D)
    blv_g = p['blv_g'].reshape(1, OUT_D)

    eps_d = jax.random.normal(jax.random.key(42), (N_DIS, OUT_D), _F32)
    eps_g = jax.random.normal(jax.random.key(43), (N_GENE, OUT_D), _F32)
    eps_g = jnp.concatenate(
        [eps_g, jnp.zeros((NG_PAD - N_GENE, OUT_D), _F32)])

    wg1 = jnp.concatenate([p['W1gd_l'], p['W1dg_r']], axis=1)
    wd1 = jnp.concatenate([p['W1dg_l'], p['W1gd_r']], axis=1)
    pg1, rg1 = _proj_pair(xg, wg1, HID, 1024)
    pd1, rd1 = _proj_pair(xd, wd1, HID, 1024)

    cntg, cntd = _sc_count_pass(src2d, dst2d, 4)
    aggg1, aggd1 = _sc_agg_pass(src2d, dst2d, pd1, pg1, HID, 2, 32, CH)

    wg2 = jnp.concatenate([p['W2gd_l'], p['W2dg_r']], axis=1)
    wd2 = jnp.concatenate([p['W2dg_l'], p['W2gd_r']], axis=1)
    pg2, rg2 = _sage_epilogue(aggg1, cntg, rg1, b1dg, wg2, LAT, 1024)
    pd2, rd2 = _sage_epilogue(aggd1, cntd, rd1, b1gd, wd2, LAT, 1024)

    src512 = src2d.reshape(E_PAD // 512, 512)
    dst512 = dst2d.reshape(E_PAD // 512, 512)
    aggg2, aggd2 = _sc_agg_pass(src512, dst512, pd2, pg2, LAT, 2, 16, 512)

    zd = _latent_d(aggd2, cntd, rd2, b2gd, p['Wmu_d'], bmu_d,
                   p['Wlv_d'], blv_d, eps_d)
    return _decode(zd, aggg2, cntg, rg2, b2dg, p['Wmu_g'], bmu_g,
                   p['Wlv_g'], blv_g, eps_g, 1024)


# speedup vs baseline: 1.1675x; 1.1675x over previous
import functools

import jax
import jax.numpy as jnp
from jax import lax
from jax.experimental import pallas as pl
from jax.experimental.pallas import tpu as pltpu
from jax.experimental.pallas import tpu_sc as plsc

N_DIS = 2000
N_GENE = 20000
E = 500000
D_IN = 128
HID = 64
LAT = 32
OUT_D = 64

NC = 2
NS = 16
NW = NC * NS
CH = 128
CPW = 128
E_PAD = NW * CPW * CH
ND_PAD = 2048
NG_PAD = 20096
GR_D = ND_PAD // NS
GR_G = NG_PAD // NS

_F32 = jnp.float32


def _zero_fill(buf, rows, width):
    @pl.loop(0, rows)
    def _zrow(i):
        @pl.loop(0, width // 16)
        def _zcol(j):
            buf[i, pl.ds(j * 16, 16)] = jnp.zeros((16,), _F32)


def _zero_stripe(zbuf, acc_sp, s, gr):
    full, rem = gr // CH, gr % CH
    @pl.loop(0, full)
    def _z(t):
        pltpu.sync_copy(zbuf, acc_sp.at[pl.ds(s * gr + t * CH, CH)])
    if rem:
        pltpu.sync_copy(zbuf.at[pl.ds(0, rem)],
                        acc_sp.at[pl.ds(s * gr + full * CH, rem)])


def _sc_agg_pass(src2d, dst2d, tbl_d, tbl_g, width, nbuf, ist, ch):
    mesh = plsc.VectorSubcoreMesh(core_axis_name="c", subcore_axis_name="s")
    ncw = (E_PAD // NW) // ch
    out_type = [
        jax.ShapeDtypeStruct((NC, NG_PAD, width), _F32),
        jax.ShapeDtypeStruct((NC, ND_PAD, width), _F32),
    ]
    scratch = (
        [pltpu.VMEM_SHARED((NG_PAD, width), _F32),
         pltpu.VMEM_SHARED((ND_PAD, width), _F32),
         pltpu.VMEM((ist, ch), jnp.int32),
         pltpu.VMEM((ist, ch), jnp.int32)]
        + [pltpu.VMEM((ch, width), _F32)] * (2 * nbuf)
        + [pltpu.SemaphoreType.DMA] * (2 * nbuf)
    )

    def body(src_hbm, dst_hbm, tbl_d_hbm, tbl_g_hbm, agg_g_hbm, agg_d_hbm,
             agg_g_sp, agg_d_sp, idx_s, idx_d, *rest):
        rows_d = rest[0:nbuf]
        rows_g = rest[nbuf:2 * nbuf]
        sem_d = rest[2 * nbuf:3 * nbuf]
        sem_g = rest[3 * nbuf:4 * nbuf]

        c = lax.axis_index("c")
        s = lax.axis_index("s")
        w = c * NS + s

        zwin = rows_d[0].at[pl.ds(0, CH)]
        _zero_fill(zwin, CH, width)
        _zero_stripe(zwin, agg_g_sp, s, GR_G)
        _zero_stripe(zwin, agg_d_sp, s, GR_D)
        plsc.subcore_barrier()

        for st in range(ncw // ist):
            base = w * ncw + st * ist
            pltpu.sync_copy(src_hbm.at[pl.ds(base, ist)], idx_s)
            pltpu.sync_copy(dst_hbm.at[pl.ds(base, ist)], idx_d)

            for b in range(nbuf):
                pltpu.async_copy(tbl_d_hbm.at[idx_s.at[b]],
                                 rows_d[b], sem_d[b])
                pltpu.async_copy(tbl_g_hbm.at[idx_d.at[b]],
                                 rows_g[b], sem_g[b])

            @pl.loop(0, ist, step=nbuf)
            def _chunk(j):
                for b in range(nbuf):
                    k = j + b
                    pltpu.make_async_copy(
                        tbl_d_hbm.at[idx_s.at[k]], rows_d[b], sem_d[b]).wait()
                    pltpu.sync_copy(rows_d[b], agg_g_sp.at[idx_d.at[k]],
                                    add=True)

                    def _refill_d(b=b, k=k):
                        pltpu.async_copy(tbl_d_hbm.at[idx_s.at[k + nbuf]],
                                         rows_d[b], sem_d[b])
                    pl.when(k + nbuf < ist)(_refill_d)

                    pltpu.make_async_copy(
                        tbl_g_hbm.at[idx_d.at[k]], rows_g[b], sem_g[b]).wait()
                    pltpu.sync_copy(rows_g[b], agg_d_sp.at[idx_s.at[k]],
                                    add=True)

                    def _refill_g(b=b, k=k):
                        pltpu.async_copy(tbl_g_hbm.at[idx_d.at[k + nbuf]],
                                         rows_g[b], sem_g[b])
                    pl.when(k + nbuf < ist)(_refill_g)

        plsc.subcore_barrier()

        pltpu.sync_copy(agg_g_sp.at[pl.ds(s * GR_G, GR_G)],
                        agg_g_hbm.at[c, pl.ds(s * GR_G, GR_G)])
        pltpu.sync_copy(agg_d_sp.at[pl.ds(s * GR_D, GR_D)],
                        agg_d_hbm.at[c, pl.ds(s * GR_D, GR_D)])

    fn = pl.kernel(
        body, out_type=out_type, mesh=mesh, scratch_types=scratch,
        compiler_params=pltpu.CompilerParams(use_tc_tiling_on_sc=False))
    return fn(src2d, dst2d, tbl_d, tbl_g)


def _sc_count_pass(src2d, dst2d, nbuf):
    mesh = plsc.VectorSubcoreMesh(core_axis_name="c", subcore_axis_name="s")
    out_type = [
        jax.ShapeDtypeStruct((NC, NG_PAD, 16), _F32),
        jax.ShapeDtypeStruct((NC, ND_PAD, 16), _F32),
    ]
    scratch = (
        [pltpu.VMEM_SHARED((NG_PAD, 16), _F32),
         pltpu.VMEM_SHARED((ND_PAD, 16), _F32),
         pltpu.VMEM((CPW, CH), jnp.int32),
         pltpu.VMEM((CPW, CH), jnp.int32),
         pltpu.VMEM((CH, 16), _F32),
         pltpu.VMEM((CH, 16), _F32)]
        + [pltpu.SemaphoreType.DMA] * (2 * nbuf)
    )

    def body(src_hbm, dst_hbm, cnt_g_hbm, cnt_d_hbm,
             cnt_g_sp, cnt_d_sp, idx_s, idx_d, ones_v, zer16_v, *sems):
        sem_g = sems[0:nbuf]
        sem_d = sems[nbuf:2 * nbuf]

        c = lax.axis_index("c")
        s = lax.axis_index("s")
        w = c * NS + s

        _zero_fill(zer16_v, CH, 16)
        @pl.loop(0, CH)
        def _orow(i):
            ones_v[i, pl.ds(0, 16)] = jnp.ones((16,), _F32)

        _zero_stripe(zer16_v, cnt_g_sp, s, GR_G)
        _zero_stripe(zer16_v, cnt_d_sp, s, GR_D)
        plsc.subcore_barrier()

        pltpu.sync_copy(src_hbm.at[pl.ds(w * CPW, CPW)], idx_s)
        pltpu.sync_copy(dst_hbm.at[pl.ds(w * CPW, CPW)], idx_d)

        @pl.loop(0, CPW, step=nbuf)
        def _chunk(j):
            for b in range(nbuf):
                k = j + b

                def _drain(b=b, k=k):
                    pltpu.make_async_copy(
                        ones_v, cnt_g_sp.at[idx_d.at[k - nbuf]],
                        sem_g[b]).wait()
                    pltpu.make_async_copy(
                        ones_v, cnt_d_sp.at[idx_s.at[k - nbuf]],
                        sem_d[b]).wait()
                pl.when(j > 0)(_drain)

                pltpu.async_copy(ones_v, cnt_g_sp.at[idx_d.at[k]],
                                 sem_g[b], add=True)
                pltpu.async_copy(ones_v, cnt_d_sp.at[idx_s.at[k]],
                                 sem_d[b], add=True)

        for b in range(nbuf):
            k = CPW - nbuf + b
            pltpu.make_async_copy(
                ones_v, cnt_g_sp.at[idx_d.at[k]], sem_g[b]).wait()
            pltpu.make_async_copy(
                ones_v, cnt_d_sp.at[idx_s.at[k]], sem_d[b]).wait()

        plsc.subcore_barrier()

        pltpu.sync_copy(cnt_g_sp.at[pl.ds(s * GR_G, GR_G)],
                        cnt_g_hbm.at[c, pl.ds(s * GR_G, GR_G)])
        pltpu.sync_copy(cnt_d_sp.at[pl.ds(s * GR_D, GR_D)],
                        cnt_d_hbm.at[c, pl.ds(s * GR_D, GR_D)])

    fn = pl.kernel(
        body, out_type=out_type, mesh=mesh, scratch_types=scratch,
        compiler_params=pltpu.CompilerParams(use_tc_tiling_on_sc=False))
    return fn(src2d, dst2d)


def _dot(a, b):
    return jnp.dot(a, b, preferred_element_type=_F32,
                   precision=lax.Precision.HIGHEST)


def _split_mm_body(x_ref, w_ref, o1_ref, o2_ref, *, half):
    acc = _dot(x_ref[...], w_ref[...])
    o1_ref[...] = acc[:, :half]
    o2_ref[...] = acc[:, half:]


def _proj_pair(x, w_cat, half, blk):
    n, d = x.shape
    return pl.pallas_call(
        functools.partial(_split_mm_body, half=half),
        grid=(pl.cdiv(n, blk),),
        in_specs=[pl.BlockSpec((blk, d), lambda i: (i, 0)),
                  pl.BlockSpec(w_cat.shape, lambda i: (0, 0))],
        out_specs=[pl.BlockSpec((blk, half), lambda i: (i, 0)),
                   pl.BlockSpec((blk, half), lambda i: (i, 0))],
        out_shape=[jax.ShapeDtypeStruct((n, half), _F32)] * 2,
    )(x, w_cat)


def _sage_epilogue_body(a_ref, c_ref, r_ref, b_ref, w_ref, o1_ref, o2_ref,
                        *, half):
    cnt = c_ref[0, :, 0:1] + c_ref[1, :, 0:1]
    agg = (a_ref[0, :, :] + a_ref[1, :, :]) / jnp.clip(cnt, 1.0, None)
    h = agg + b_ref[...] + r_ref[...]
    acc = _dot(h, w_ref[...])
    o1_ref[...] = acc[:, :half]
    o2_ref[...] = acc[:, half:]


def _sage_epilogue(agg_p, cnt_p, r, b, w_cat, half, blk):
    n, hid = r.shape
    return pl.pallas_call(
        functools.partial(_sage_epilogue_body, half=half),
        grid=(pl.cdiv(n, blk),),
        in_specs=[pl.BlockSpec((2, blk, hid), lambda i: (0, i, 0)),
                  pl.BlockSpec((2, blk, 16), lambda i: (0, i, 0)),
                  pl.BlockSpec((blk, hid), lambda i: (i, 0)),
                  pl.BlockSpec(b.shape, lambda i: (0, 0)),
                  pl.BlockSpec(w_cat.shape, lambda i: (0, 0))],
        out_specs=[pl.BlockSpec((blk, half), lambda i: (i, 0)),
                   pl.BlockSpec((blk, half), lambda i: (i, 0))],
        out_shape=[jax.ShapeDtypeStruct((n, half), _F32)] * 2,
    )(agg_p, cnt_p, r, b, w_cat)


def _latent_body(a_ref, c_ref, r_ref, b_ref, wmu_ref, bmu_ref, wlv_ref,
                 blv_ref, eps_ref, o_ref):
    cnt = c_ref[0, :, 0:1] + c_ref[1, :, 0:1]
    h = ((a_ref[0, :, :] + a_ref[1, :, :]) / jnp.clip(cnt, 1.0, None)
         + b_ref[...] + r_ref[...])
    mu = _dot(h, wmu_ref[...]) + bmu_ref[...]
    lv = _dot(h, wlv_ref[...]) + blv_ref[...]
    o_ref[...] = mu + eps_ref[...] * jnp.exp(lv)


def _latent_d(agg_p, cnt_p, r, b, wmu, bmu, wlv, blv, eps):
    return pl.pallas_call(
        _latent_body,
        grid=(1,),
        in_specs=[pl.BlockSpec((2, N_DIS, LAT), lambda i: (0, 0, 0)),
                  pl.BlockSpec((2, N_DIS, 16), lambda i: (0, 0, 0)),
                  pl.BlockSpec((N_DIS, LAT), lambda i: (0, 0)),
                  pl.BlockSpec(b.shape, lambda i: (0, 0)),
                  pl.BlockSpec(wmu.shape, lambda i: (0, 0)),
                  pl.BlockSpec(bmu.shape, lambda i: (0, 0)),
                  pl.BlockSpec(wlv.shape, lambda i: (0, 0)),
                  pl.BlockSpec(blv.shape, lambda i: (0, 0)),
                  pl.BlockSpec(eps.shape, lambda i: (0, 0))],
        out_specs=pl.BlockSpec((N_DIS, OUT_D), lambda i: (0, 0)),
        out_shape=jax.ShapeDtypeStruct((N_DIS, OUT_D), _F32),
    )(agg_p, cnt_p, r, b, wmu, bmu, wlv, blv, eps)


def _decode_body(zd_ref, a_ref, c_ref, r_ref, b_ref, wmu_ref, bmu_ref,
                 wlv_ref, blv_ref, eps_ref, o_ref):
    cnt = c_ref[0, :, 0:1] + c_ref[1, :, 0:1]
    h = ((a_ref[0, :, :] + a_ref[1, :, :]) / jnp.clip(cnt, 1.0, None)
         + b_ref[...] + r_ref[...])
    mu = _dot(h, wmu_ref[...]) + bmu_ref[...]
    lv = _dot(h, wlv_ref[...]) + blv_ref[...]
    zg = mu + eps_ref[...] * jnp.exp(lv)
    zd = zd_ref[...]
    dh = zd.astype(jnp.bfloat16)
    dl = (zd - dh.astype(_F32)).astype(jnp.bfloat16)
    gh = zg.astype(jnp.bfloat16)
    gl = (zg - gh.astype(_F32)).astype(jnp.bfloat16)
    dn = (((1,), (1,)), ((), ()))
    hi = lax.dot_general(jnp.concatenate([dh, dl], axis=1),
                         jnp.concatenate([gh, gh], axis=1), dn,
                         preferred_element_type=_F32)
    lo = lax.dot_general(dh, gl, dn, preferred_element_type=_F32)
    o_ref[...] = hi + lo


def _decode(zd, agg_p, cnt_p, r, b, wmu, bmu, wlv, blv, eps, blk):
    return pl.pallas_call(
        _decode_body,
        grid=(pl.cdiv(NG_PAD, blk),),
        in_specs=[pl.BlockSpec((N_DIS, OUT_D), lambda i: (0, 0)),
                  pl.BlockSpec((2, blk, LAT), lambda i: (0, i, 0)),
                  pl.BlockSpec((2, blk, 16), lambda i: (0, i, 0)),
                  pl.BlockSpec((blk, LAT), lambda i: (i, 0)),
                  pl.BlockSpec(b.shape, lambda i: (0, 0)),
                  pl.BlockSpec(wmu.shape, lambda i: (0, 0)),
                  pl.BlockSpec(bmu.shape, lambda i: (0, 0)),
                  pl.BlockSpec(wlv.shape, lambda i: (0, 0)),
                  pl.BlockSpec(blv.shape, lambda i: (0, 0)),
                  pl.BlockSpec((blk, OUT_D), lambda i: (i, 0))],
        out_specs=pl.BlockSpec((N_DIS, blk), lambda i: (0, i)),
        out_shape=jax.ShapeDtypeStruct((N_DIS, N_GENE), _F32),
    )(zd, agg_p, cnt_p, r, b, wmu, bmu, wlv, blv, eps)


def kernel(x_disease, x_gene, edge_src, edge_dst, params):
    p = params

    src = edge_src.astype(jnp.int32)
    dst = edge_dst.astype(jnp.int32)
    pad_i = jnp.arange(E_PAD - E, dtype=jnp.int32)
    src2d = jnp.concatenate(
        [src, N_DIS + pad_i % (ND_PAD - N_DIS)]).reshape(NW * CPW, CH)
    dst2d = jnp.concatenate(
        [dst, N_GENE + pad_i % (NG_PAD - N_GENE)]).reshape(NW * CPW, CH)
    xd = jnp.concatenate(
        [x_disease, jnp.zeros((ND_PAD - N_DIS, D_IN), _F32)])
    xg = jnp.concatenate(
        [x_gene, jnp.zeros((NG_PAD - N_GENE, D_IN), _F32)])

    b1dg = p['b1dg'].reshape(1, HID)
    b1gd = p['b1gd'].reshape(1, HID)
    b2dg = p['b2dg'].reshape(1, LAT)
    b2gd = p['b2gd'].reshape(1, LAT)
    bmu_d = p['bmu_d'].reshape(1, OUT_D)
    blv_d = p['blv_d'].reshape(1, OUT_D)
    bmu_g = p['bmu_g'].reshape(1, OUT_D)
    blv_g = p['blv_g'].reshape(1, OUT_D)

    eps_d = jax.random.normal(jax.random.key(42), (N_DIS, OUT_D), _F32)
    eps_g = jax.random.normal(jax.random.key(43), (N_GENE, OUT_D), _F32)
    eps_g = jnp.concatenate(
        [eps_g, jnp.zeros((NG_PAD - N_GENE, OUT_D), _F32)])

    wg1 = jnp.concatenate([p['W1gd_l'], p['W1dg_r']], axis=1)
    wd1 = jnp.concatenate([p['W1dg_l'], p['W1gd_r']], axis=1)
    pg1, rg1 = _proj_pair(xg, wg1, HID, 1024)
    pd1, rd1 = _proj_pair(xd, wd1, HID, 1024)

    cntg, cntd = _sc_count_pass(src2d, dst2d, 4)
    aggg1, aggd1 = _sc_agg_pass(src2d, dst2d, pd1, pg1, HID, 2, 32, CH)

    wg2 = jnp.concatenate([p['W2gd_l'], p['W2dg_r']], axis=1)
    wd2 = jnp.concatenate([p['W2dg_l'], p['W2gd_r']], axis=1)
    pg2, rg2 = _sage_epilogue(aggg1, cntg, rg1, b1dg, wg2, LAT, 1024)
    pd2, rd2 = _sage_epilogue(aggd1, cntd, rd1, b1gd, wd2, LAT, 1024)

    aggg2, aggd2 = _sc_agg_pass(src2d, dst2d, pd2, pg2, LAT, 4, CPW, CH)

    zd = _latent_d(aggd2, cntd, rd2, b2gd, p['Wmu_d'], bmu_d,
                   p['Wlv_d'], blv_d, eps_d)
    return _decode(zd, aggg2, cntg, rg2, b2dg, p['Wmu_g'], bmu_g,
                   p['Wlv_g'], blv_g, eps_g, 1024)


# speedup vs baseline: 1.1690x; 1.0013x over previous
import functools

import jax
import jax.numpy as jnp
from jax import lax
from jax.experimental import pallas as pl
from jax.experimental.pallas import tpu as pltpu
from jax.experimental.pallas import tpu_sc as plsc

N_DIS = 2000
N_GENE = 20000
E = 500000
D_IN = 128
HID = 64
LAT = 32
OUT_D = 64

NC = 2
NS = 16
NW = NC * NS
CH = 128
CPW = 128
E_PAD = NW * CPW * CH
ND_PAD = 2048
NG_PAD = 20096
GR_D = ND_PAD // NS
GR_G = NG_PAD // NS

_F32 = jnp.float32


def _zero_fill(buf, rows, width):
    @pl.loop(0, rows)
    def _zrow(i):
        @pl.loop(0, width // 16)
        def _zcol(j):
            buf[i, pl.ds(j * 16, 16)] = jnp.zeros((16,), _F32)


def _zero_stripe(zbuf, acc_sp, s, gr):
    full, rem = gr // CH, gr % CH
    @pl.loop(0, full)
    def _z(t):
        pltpu.sync_copy(zbuf, acc_sp.at[pl.ds(s * gr + t * CH, CH)])
    if rem:
        pltpu.sync_copy(zbuf.at[pl.ds(0, rem)],
                        acc_sp.at[pl.ds(s * gr + full * CH, rem)])


def _sc_agg_pass(src2d, dst2d, tbl_d, tbl_g, width, nbuf, ist, ch):
    mesh = plsc.VectorSubcoreMesh(core_axis_name="c", subcore_axis_name="s")
    ncw = (E_PAD // NW) // ch
    out_type = [
        jax.ShapeDtypeStruct((NC, NG_PAD, width), _F32),
        jax.ShapeDtypeStruct((NC, ND_PAD, width), _F32),
    ]
    scratch = (
        [pltpu.VMEM_SHARED((NG_PAD, width), _F32),
         pltpu.VMEM_SHARED((ND_PAD, width), _F32),
         pltpu.VMEM((ist, ch), jnp.int32),
         pltpu.VMEM((ist, ch), jnp.int32)]
        + [pltpu.VMEM((ch, width), _F32)] * (2 * nbuf)
        + [pltpu.SemaphoreType.DMA] * (2 * nbuf)
    )

    def body(src_hbm, dst_hbm, tbl_d_hbm, tbl_g_hbm, agg_g_hbm, agg_d_hbm,
             agg_g_sp, agg_d_sp, idx_s, idx_d, *rest):
        rows_d = rest[0:nbuf]
        rows_g = rest[nbuf:2 * nbuf]
        sem_d = rest[2 * nbuf:3 * nbuf]
        sem_g = rest[3 * nbuf:4 * nbuf]

        c = lax.axis_index("c")
        s = lax.axis_index("s")
        w = c * NS + s

        zwin = rows_d[0].at[pl.ds(0, CH)]
        _zero_fill(zwin, CH, width)
        _zero_stripe(zwin, agg_g_sp, s, GR_G)
        _zero_stripe(zwin, agg_d_sp, s, GR_D)
        plsc.subcore_barrier()

        for st in range(ncw // ist):
            base = w * ncw + st * ist
            pltpu.sync_copy(src_hbm.at[pl.ds(base, ist)], idx_s)
            pltpu.sync_copy(dst_hbm.at[pl.ds(base, ist)], idx_d)

            for b in range(nbuf):
                pltpu.async_copy(tbl_d_hbm.at[idx_s.at[b]],
                                 rows_d[b], sem_d[b])
                pltpu.async_copy(tbl_g_hbm.at[idx_d.at[b]],
                                 rows_g[b], sem_g[b])

            @pl.loop(0, ist, step=nbuf)
            def _chunk(j):
                for b in range(nbuf):
                    k = j + b
                    pltpu.make_async_copy(
                        tbl_d_hbm.at[idx_s.at[k]], rows_d[b], sem_d[b]).wait()
                    pltpu.sync_copy(rows_d[b], agg_g_sp.at[idx_d.at[k]],
                                    add=True)

                    def _refill_d(b=b, k=k):
                        pltpu.async_copy(tbl_d_hbm.at[idx_s.at[k + nbuf]],
                                         rows_d[b], sem_d[b])
                    pl.when(k + nbuf < ist)(_refill_d)

                    pltpu.make_async_copy(
                        tbl_g_hbm.at[idx_d.at[k]], rows_g[b], sem_g[b]).wait()
                    pltpu.sync_copy(rows_g[b], agg_d_sp.at[idx_s.at[k]],
                                    add=True)

                    def _refill_g(b=b, k=k):
                        pltpu.async_copy(tbl_g_hbm.at[idx_d.at[k + nbuf]],
                                         rows_g[b], sem_g[b])
                    pl.when(k + nbuf < ist)(_refill_g)

        plsc.subcore_barrier()

        pltpu.sync_copy(agg_g_sp.at[pl.ds(s * GR_G, GR_G)],
                        agg_g_hbm.at[c, pl.ds(s * GR_G, GR_G)])
        pltpu.sync_copy(agg_d_sp.at[pl.ds(s * GR_D, GR_D)],
                        agg_d_hbm.at[c, pl.ds(s * GR_D, GR_D)])

    fn = pl.kernel(
        body, out_type=out_type, mesh=mesh, scratch_types=scratch,
        compiler_params=pltpu.CompilerParams(use_tc_tiling_on_sc=False))
    return fn(src2d, dst2d, tbl_d, tbl_g)


def _sc_count_pass(src2d, dst2d, nbuf):
    mesh = plsc.VectorSubcoreMesh(core_axis_name="c", subcore_axis_name="s")
    out_type = [
        jax.ShapeDtypeStruct((NC, NG_PAD, 16), _F32),
        jax.ShapeDtypeStruct((NC, ND_PAD, 16), _F32),
    ]
    scratch = (
        [pltpu.VMEM_SHARED((NG_PAD, 16), _F32),
         pltpu.VMEM_SHARED((ND_PAD, 16), _F32),
         pltpu.VMEM((CPW, CH), jnp.int32),
         pltpu.VMEM((CPW, CH), jnp.int32),
         pltpu.VMEM((CH, 16), _F32),
         pltpu.VMEM((CH, 16), _F32)]
        + [pltpu.SemaphoreType.DMA] * (2 * nbuf)
    )

    def body(src_hbm, dst_hbm, cnt_g_hbm, cnt_d_hbm,
             cnt_g_sp, cnt_d_sp, idx_s, idx_d, ones_v, zer16_v, *sems):
        sem_g = sems[0:nbuf]
        sem_d = sems[nbuf:2 * nbuf]

        c = lax.axis_index("c")
        s = lax.axis_index("s")
        w = c * NS + s

        _zero_fill(zer16_v, CH, 16)
        @pl.loop(0, CH)
        def _orow(i):
            ones_v[i, pl.ds(0, 16)] = jnp.ones((16,), _F32)

        _zero_stripe(zer16_v, cnt_g_sp, s, GR_G)
        _zero_stripe(zer16_v, cnt_d_sp, s, GR_D)
        plsc.subcore_barrier()

        pltpu.sync_copy(src_hbm.at[pl.ds(w * CPW, CPW)], idx_s)
        pltpu.sync_copy(dst_hbm.at[pl.ds(w * CPW, CPW)], idx_d)

        @pl.loop(0, CPW, step=nbuf)
        def _chunk(j):
            for b in range(nbuf):
                k = j + b

                def _drain(b=b, k=k):
                    pltpu.make_async_copy(
                        ones_v, cnt_g_sp.at[idx_d.at[k - nbuf]],
                        sem_g[b]).wait()
                    pltpu.make_async_copy(
                        ones_v, cnt_d_sp.at[idx_s.at[k - nbuf]],
                        sem_d[b]).wait()
                pl.when(j > 0)(_drain)

                pltpu.async_copy(ones_v, cnt_g_sp.at[idx_d.at[k]],
                                 sem_g[b], add=True)
                pltpu.async_copy(ones_v, cnt_d_sp.at[idx_s.at[k]],
                                 sem_d[b], add=True)

        for b in range(nbuf):
            k = CPW - nbuf + b
            pltpu.make_async_copy(
                ones_v, cnt_g_sp.at[idx_d.at[k]], sem_g[b]).wait()
            pltpu.make_async_copy(
                ones_v, cnt_d_sp.at[idx_s.at[k]], sem_d[b]).wait()

        plsc.subcore_barrier()

        pltpu.sync_copy(cnt_g_sp.at[pl.ds(s * GR_G, GR_G)],
                        cnt_g_hbm.at[c, pl.ds(s * GR_G, GR_G)])
        pltpu.sync_copy(cnt_d_sp.at[pl.ds(s * GR_D, GR_D)],
                        cnt_d_hbm.at[c, pl.ds(s * GR_D, GR_D)])

    fn = pl.kernel(
        body, out_type=out_type, mesh=mesh, scratch_types=scratch,
        compiler_params=pltpu.CompilerParams(use_tc_tiling_on_sc=False))
    return fn(src2d, dst2d)


def _dot(a, b):
    return jnp.dot(a, b, preferred_element_type=_F32,
                   precision=lax.Precision.HIGHEST)


def _split_mm_body(x_ref, w_ref, o1_ref, o2_ref, *, half):
    acc = _dot(x_ref[...], w_ref[...])
    o1_ref[...] = acc[:, :half]
    o2_ref[...] = acc[:, half:]


def _proj_pair(x, w_cat, half, blk):
    n, d = x.shape
    return pl.pallas_call(
        functools.partial(_split_mm_body, half=half),
        grid=(pl.cdiv(n, blk),),
        in_specs=[pl.BlockSpec((blk, d), lambda i: (i, 0)),
                  pl.BlockSpec(w_cat.shape, lambda i: (0, 0))],
        out_specs=[pl.BlockSpec((blk, half), lambda i: (i, 0)),
                   pl.BlockSpec((blk, half), lambda i: (i, 0))],
        out_shape=[jax.ShapeDtypeStruct((n, half), _F32)] * 2,
    )(x, w_cat)


def _mm_body(x_ref, w_ref, o_ref):
    o_ref[...] = _dot(x_ref[...], w_ref[...])


def _mm(x, w, blk):
    n, d = x.shape
    return pl.pallas_call(
        _mm_body,
        grid=(pl.cdiv(n, blk),),
        in_specs=[pl.BlockSpec((blk, d), lambda i: (i, 0)),
                  pl.BlockSpec(w.shape, lambda i: (0, 0))],
        out_specs=pl.BlockSpec((blk, w.shape[1]), lambda i: (i, 0)),
        out_shape=jax.ShapeDtypeStruct((n, w.shape[1]), _F32),
    )(x, w)


def _sage_h_body(a_ref, c_ref, r_ref, b_ref, w_ref, o1_ref, o2_ref):
    cnt = c_ref[0, :, 0:1] + c_ref[1, :, 0:1]
    agg = (a_ref[0, :, :] + a_ref[1, :, :]) / jnp.clip(cnt, 1.0, None)
    h = agg + b_ref[...] + r_ref[...]
    o1_ref[...] = _dot(h, w_ref[...])
    o2_ref[...] = h


def _sage_epilogue_h(agg_p, cnt_p, r, b, w_l, blk):
    n, hid = r.shape
    return pl.pallas_call(
        _sage_h_body,
        grid=(pl.cdiv(n, blk),),
        in_specs=[pl.BlockSpec((2, blk, hid), lambda i: (0, i, 0)),
                  pl.BlockSpec((2, blk, 16), lambda i: (0, i, 0)),
                  pl.BlockSpec((blk, hid), lambda i: (i, 0)),
                  pl.BlockSpec(b.shape, lambda i: (0, 0)),
                  pl.BlockSpec(w_l.shape, lambda i: (0, 0))],
        out_specs=[pl.BlockSpec((blk, w_l.shape[1]), lambda i: (i, 0)),
                   pl.BlockSpec((blk, hid), lambda i: (i, 0))],
        out_shape=[jax.ShapeDtypeStruct((n, w_l.shape[1]), _F32),
                   jax.ShapeDtypeStruct((n, hid), _F32)],
    )(agg_p, cnt_p, r, b, w_l)


def _sage_epilogue_body(a_ref, c_ref, r_ref, b_ref, w_ref, o1_ref, o2_ref,
                        *, half):
    cnt = c_ref[0, :, 0:1] + c_ref[1, :, 0:1]
    agg = (a_ref[0, :, :] + a_ref[1, :, :]) / jnp.clip(cnt, 1.0, None)
    h = agg + b_ref[...] + r_ref[...]
    acc = _dot(h, w_ref[...])
    o1_ref[...] = acc[:, :half]
    o2_ref[...] = acc[:, half:]


def _sage_epilogue(agg_p, cnt_p, r, b, w_cat, half, blk):
    n, hid = r.shape
    return pl.pallas_call(
        functools.partial(_sage_epilogue_body, half=half),
        grid=(pl.cdiv(n, blk),),
        in_specs=[pl.BlockSpec((2, blk, hid), lambda i: (0, i, 0)),
                  pl.BlockSpec((2, blk, 16), lambda i: (0, i, 0)),
                  pl.BlockSpec((blk, hid), lambda i: (i, 0)),
                  pl.BlockSpec(b.shape, lambda i: (0, 0)),
                  pl.BlockSpec(w_cat.shape, lambda i: (0, 0))],
        out_specs=[pl.BlockSpec((blk, half), lambda i: (i, 0)),
                   pl.BlockSpec((blk, half), lambda i: (i, 0))],
        out_shape=[jax.ShapeDtypeStruct((n, half), _F32)] * 2,
    )(agg_p, cnt_p, r, b, w_cat)


def _latent_body(a_ref, c_ref, r_ref, b_ref, wmu_ref, bmu_ref, wlv_ref,
                 blv_ref, eps_ref, o_ref):
    cnt = c_ref[0, :, 0:1] + c_ref[1, :, 0:1]
    h = ((a_ref[0, :, :] + a_ref[1, :, :]) / jnp.clip(cnt, 1.0, None)
         + b_ref[...] + r_ref[...])
    mu = _dot(h, wmu_ref[...]) + bmu_ref[...]
    lv = _dot(h, wlv_ref[...]) + blv_ref[...]
    o_ref[...] = mu + eps_ref[...] * jnp.exp(lv)


def _latent_d(agg_p, cnt_p, r, b, wmu, bmu, wlv, blv, eps):
    return pl.pallas_call(
        _latent_body,
        grid=(1,),
        in_specs=[pl.BlockSpec((2, N_DIS, LAT), lambda i: (0, 0, 0)),
                  pl.BlockSpec((2, N_DIS, 16), lambda i: (0, 0, 0)),
                  pl.BlockSpec((N_DIS, LAT), lambda i: (0, 0)),
                  pl.BlockSpec(b.shape, lambda i: (0, 0)),
                  pl.BlockSpec(wmu.shape, lambda i: (0, 0)),
                  pl.BlockSpec(bmu.shape, lambda i: (0, 0)),
                  pl.BlockSpec(wlv.shape, lambda i: (0, 0)),
                  pl.BlockSpec(blv.shape, lambda i: (0, 0)),
                  pl.BlockSpec(eps.shape, lambda i: (0, 0))],
        out_specs=pl.BlockSpec((N_DIS, OUT_D), lambda i: (0, 0)),
        out_shape=jax.ShapeDtypeStruct((N_DIS, OUT_D), _F32),
    )(agg_p, cnt_p, r, b, wmu, bmu, wlv, blv, eps)


def _decode_body(zd_ref, a_ref, c_ref, r_ref, b_ref, wmu_ref, bmu_ref,
                 wlv_ref, blv_ref, eps_ref, o_ref):
    cnt = c_ref[0, :, 0:1] + c_ref[1, :, 0:1]
    h = ((a_ref[0, :, :] + a_ref[1, :, :]) / jnp.clip(cnt, 1.0, None)
         + b_ref[...] + r_ref[...])
    mu = _dot(h, wmu_ref[...]) + bmu_ref[...]
    lv = _dot(h, wlv_ref[...]) + blv_ref[...]
    zg = mu + eps_ref[...] * jnp.exp(lv)
    zd = zd_ref[...]
    dh = zd.astype(jnp.bfloat16)
    dl = (zd - dh.astype(_F32)).astype(jnp.bfloat16)
    gh = zg.astype(jnp.bfloat16)
    gl = (zg - gh.astype(_F32)).astype(jnp.bfloat16)
    dn = (((1,), (1,)), ((), ()))
    hi = lax.dot_general(jnp.concatenate([dh, dl], axis=1),
                         jnp.concatenate([gh, gh], axis=1), dn,
                         preferred_element_type=_F32)
    lo = lax.dot_general(dh, gl, dn, preferred_element_type=_F32)
    o_ref[...] = hi + lo


def _decode(zd, agg_p, cnt_p, r, b, wmu, bmu, wlv, blv, eps, blk):
    return pl.pallas_call(
        _decode_body,
        grid=(pl.cdiv(NG_PAD, blk),),
        in_specs=[pl.BlockSpec((N_DIS, OUT_D), lambda i: (0, 0)),
                  pl.BlockSpec((2, blk, LAT), lambda i: (0, i, 0)),
                  pl.BlockSpec((2, blk, 16), lambda i: (0, i, 0)),
                  pl.BlockSpec((blk, LAT), lambda i: (i, 0)),
                  pl.BlockSpec(b.shape, lambda i: (0, 0)),
                  pl.BlockSpec(wmu.shape, lambda i: (0, 0)),
                  pl.BlockSpec(bmu.shape, lambda i: (0, 0)),
                  pl.BlockSpec(wlv.shape, lambda i: (0, 0)),
                  pl.BlockSpec(blv.shape, lambda i: (0, 0)),
                  pl.BlockSpec((blk, OUT_D), lambda i: (i, 0))],
        out_specs=pl.BlockSpec((N_DIS, blk), lambda i: (0, i)),
        out_shape=jax.ShapeDtypeStruct((N_DIS, N_GENE), _F32),
    )(zd, agg_p, cnt_p, r, b, wmu, bmu, wlv, blv, eps)


def kernel(x_disease, x_gene, edge_src, edge_dst, params):
    p = params

    src = edge_src.astype(jnp.int32)
    dst = edge_dst.astype(jnp.int32)
    pad_i = jnp.arange(E_PAD - E, dtype=jnp.int32)
    src2d = jnp.concatenate(
        [src, N_DIS + pad_i % (ND_PAD - N_DIS)]).reshape(NW * CPW, CH)
    dst2d = jnp.concatenate(
        [dst, N_GENE + pad_i % (NG_PAD - N_GENE)]).reshape(NW * CPW, CH)
    xd = jnp.concatenate(
        [x_disease, jnp.zeros((ND_PAD - N_DIS, D_IN), _F32)])
    xg = jnp.concatenate(
        [x_gene, jnp.zeros((NG_PAD - N_GENE, D_IN), _F32)])

    b1dg = p['b1dg'].reshape(1, HID)
    b1gd = p['b1gd'].reshape(1, HID)
    b2dg = p['b2dg'].reshape(1, LAT)
    b2gd = p['b2gd'].reshape(1, LAT)
    bmu_d = p['bmu_d'].reshape(1, OUT_D)
    blv_d = p['blv_d'].reshape(1, OUT_D)
    bmu_g = p['bmu_g'].reshape(1, OUT_D)
    blv_g = p['blv_g'].reshape(1, OUT_D)

    eps_d = jax.random.normal(jax.random.key(42), (N_DIS, OUT_D), _F32)
    eps_g = jax.random.normal(jax.random.key(43), (N_GENE, OUT_D), _F32)
    eps_g = jnp.concatenate(
        [eps_g, jnp.zeros((NG_PAD - N_GENE, OUT_D), _F32)])

    wd1 = jnp.concatenate([p['W1dg_l'], p['W1gd_r']], axis=1)
    pg1 = _mm(xg, p['W1gd_l'], 1024)
    rg1 = _mm(xg, p['W1dg_r'], 1024)
    pd1, rd1 = _proj_pair(xd, wd1, HID, 1024)

    cntg, cntd = _sc_count_pass(src2d, dst2d, 4)
    aggg1, aggd1 = _sc_agg_pass(src2d, dst2d, pd1, pg1, HID, 2, 32, CH)

    wd2 = jnp.concatenate([p['W2dg_l'], p['W2gd_r']], axis=1)
    pg2, h_g = _sage_epilogue_h(aggg1, cntg, rg1, b1dg, p['W2gd_l'], 1024)
    rg2 = _mm(h_g, p['W2dg_r'], 1024)
    pd2, rd2 = _sage_epilogue(aggd1, cntd, rd1, b1gd, wd2, LAT, 1024)

    aggg2, aggd2 = _sc_agg_pass(src2d, dst2d, pd2, pg2, LAT, 4, CPW, CH)

    zd = _latent_d(aggd2, cntd, rd2, b2gd, p['Wmu_d'], bmu_d,
                   p['Wlv_d'], blv_d, eps_d)
    return _decode(zd, aggg2, cntg, rg2, b2dg, p['Wmu_g'], bmu_g,
                   p['Wlv_g'], blv_g, eps_g, 1024)
